# Initial kernel scaffold; baseline (speedup 1.0000x reference)
import jax
import jax.numpy as jnp
from jax.experimental import pallas as pl


def kernel(x, edge_index, W_enc, Wmsg, Wself, alpha, beta, ln_scale, ln_bias, W_head, b_head):
    raise NotImplementedError("write your pallas kernel here")


# speedup vs baseline: 3.3650x; 3.3650x over previous
import functools

import jax
import jax.numpy as jnp
from jax import lax
from jax.experimental import pallas as pl
from jax.experimental.pallas import tpu as pltpu
from jax.experimental.pallas import tpu_sc as plsc

N = 10000
E = 160000
D = 256
HALF = 128
L = 4

CHUNK = 128
NTILES = 16
EPAD = 163840
NCHUNK = EPAD // CHUNK
CPT = NCHUNK // NTILES
NROWS = 10240
RPT = NROWS // NTILES

BLK = 400
GRID = N // BLK


def _sc_agg_body(y0, y1, srcs, dsts, out0, out1, src_v, dst_v, rows_v, agg_sh,
                 sem):
    c = lax.axis_index("c")
    s = lax.axis_index("s")

    pltpu.sync_copy(srcs.at[pl.ds(s * CPT, CPT)], src_v)
    pltpu.sync_copy(dsts.at[pl.ds(s * CPT, CPT)], dst_v)

    def _zero_row(r, carry):
        for k in range(HALF // 16):
            rows_v[r, pl.ds(k * 16, 16)] = jnp.zeros((16,), jnp.float32)
        return carry

    lax.fori_loop(0, CHUNK, _zero_row, 0)
    for z in range(RPT // CHUNK):
        pltpu.sync_copy(rows_v, agg_sh.at[pl.ds(s * RPT + z * CHUNK, CHUNK)])
    plsc.subcore_barrier()

    def _run(y_ref):
        def _step(j, carry):
            pltpu.async_copy(y_ref.at[src_v.at[j]], rows_v, sem).wait()
            pltpu.sync_copy(rows_v, agg_sh.at[dst_v.at[j]], add=True)
            return carry
        lax.fori_loop(0, CPT, _step, 0)

    pl.when(c == 0)(lambda: _run(y0))
    pl.when(c == 1)(lambda: _run(y1))
    plsc.subcore_barrier()

    rows = pl.ds(s * RPT, RPT)
    pl.when(c == 0)(lambda: pltpu.sync_copy(agg_sh.at[rows], out0.at[rows]))
    pl.when(c == 1)(lambda: pltpu.sync_copy(agg_sh.at[rows], out1.at[rows]))


def _sc_agg(y0, y1, srcs, dsts):
    mesh = plsc.VectorSubcoreMesh(core_axis_name="c", subcore_axis_name="s")
    f = pl.kernel(
        _sc_agg_body,
        mesh=mesh,
        out_type=[
            jax.ShapeDtypeStruct((NROWS, HALF), jnp.float32),
            jax.ShapeDtypeStruct((NROWS, HALF), jnp.float32),
        ],
        scratch_types=[
            pltpu.VMEM((CPT, CHUNK), jnp.int32),
            pltpu.VMEM((CPT, CHUNK), jnp.int32),
            pltpu.VMEM((CHUNK, HALF), jnp.float32),
            pltpu.VMEM_SHARED((NROWS, HALF), jnp.float32),
            pltpu.SemaphoreType.DMA,
        ],
    )
    return f(y0, y1, srcs, dsts)


def _layernorm(h, scale, bias):
    mu = jnp.mean(h, axis=-1, keepdims=True)
    d = h - mu
    var = jnp.mean(d * d, axis=-1, keepdims=True)
    return d * lax.rsqrt(var + 1e-5) * scale + bias


def _enc_body(x_ref, wenc_ref, wmsg_ref, x1_ref, y0_ref, y1_ref):
    x1 = jnp.dot(x_ref[...], wenc_ref[...], preferred_element_type=jnp.float32)
    x1_ref[...] = x1
    y = jnp.dot(x1, wmsg_ref[...], preferred_element_type=jnp.float32)
    y0_ref[...] = y[:, :HALF]
    y1_ref[...] = y[:, HALF:]


def _combine_mid_body(x_ref, a0_ref, a1_ref, wself_ref, wnext_ref, al_ref,
                      be_ref, lns_ref, lnb_ref, xn_ref, y0_ref, y1_ref, *,
                      residual):
    x = x_ref[...]
    sv = jnp.dot(x, wself_ref[...], preferred_element_type=jnp.float32)
    agg = jnp.concatenate([a0_ref[...], a1_ref[...]], axis=1)
    h = jnp.maximum(al_ref[0, 0] * sv + be_ref[0, 0] * agg, 0.0)
    h = _layernorm(h, lns_ref[...], lnb_ref[...])
    xn = h + x if residual else h
    xn_ref[...] = xn
    y = jnp.dot(xn, wnext_ref[...], preferred_element_type=jnp.float32)
    y0_ref[...] = y[:, :HALF]
    y1_ref[...] = y[:, HALF:]


def _combine_last_body(x_ref, a0_ref, a1_ref, wself_ref, whead_ref, al_ref,
                       be_ref, lns_ref, lnb_ref, bh_ref, out_ref):
    x = x_ref[...]
    sv = jnp.dot(x, wself_ref[...], preferred_element_type=jnp.float32)
    agg = jnp.concatenate([a0_ref[...], a1_ref[...]], axis=1)
    h = jnp.maximum(al_ref[0, 0] * sv + be_ref[0, 0] * agg, 0.0)
    h = _layernorm(h, lns_ref[...], lnb_ref[...])
    xn = h + x
    out_ref[...] = (jnp.dot(xn, whead_ref[...],
                            preferred_element_type=jnp.float32) + bh_ref[...])


def _row_spec(cols):
    return pl.BlockSpec((BLK, cols), lambda i: (i, 0))


def _full_spec(r, c):
    return pl.BlockSpec((r, c), lambda i: (0, 0))


def _enc_call(x, wenc, wmsg0):
    return pl.pallas_call(
        _enc_body,
        grid=(GRID,),
        in_specs=[_row_spec(D), _full_spec(D, D), _full_spec(D, D)],
        out_specs=[_row_spec(D), _row_spec(HALF), _row_spec(HALF)],
        out_shape=[
            jax.ShapeDtypeStruct((N, D), jnp.float32),
            jax.ShapeDtypeStruct((N, HALF), jnp.float32),
            jax.ShapeDtypeStruct((N, HALF), jnp.float32),
        ],
    )(x, wenc, wmsg0)


def _combine_mid_call(x, a0, a1, wself, wnext, al, be, lns, lnb, residual):
    body = functools.partial(_combine_mid_body, residual=residual)
    return pl.pallas_call(
        body,
        grid=(GRID,),
        in_specs=[
            _row_spec(D), _row_spec(HALF), _row_spec(HALF),
            _full_spec(D, D), _full_spec(D, D),
            _full_spec(1, 1), _full_spec(1, 1),
            _full_spec(1, D), _full_spec(1, D),
        ],
        out_specs=[_row_spec(D), _row_spec(HALF), _row_spec(HALF)],
        out_shape=[
            jax.ShapeDtypeStruct((N, D), jnp.float32),
            jax.ShapeDtypeStruct((N, HALF), jnp.float32),
            jax.ShapeDtypeStruct((N, HALF), jnp.float32),
        ],
    )(x, a0, a1, wself, wnext, al, be, lns, lnb)


def _combine_last_call(x, a0, a1, wself, whead, al, be, lns, lnb, bh):
    return pl.pallas_call(
        _combine_last_body,
        grid=(GRID,),
        in_specs=[
            _row_spec(D), _row_spec(HALF), _row_spec(HALF),
            _full_spec(D, D), _full_spec(D, D),
            _full_spec(1, 1), _full_spec(1, 1),
            _full_spec(1, D), _full_spec(1, D), _full_spec(1, D),
        ],
        out_specs=_row_spec(D),
        out_shape=jax.ShapeDtypeStruct((N, D), jnp.float32),
    )(x, a0, a1, wself, whead, al, be, lns, lnb, bh)


def kernel(x, edge_index, W_enc, Wmsg, Wself, alpha, beta, ln_scale, ln_bias,
           W_head, b_head):
    src = edge_index[0].astype(jnp.int32)
    dst = edge_index[1].astype(jnp.int32)
    pad = EPAD - E
    srcs = jnp.concatenate([src, jnp.zeros((pad,), jnp.int32)])
    srcs = srcs.reshape(NCHUNK, CHUNK)
    dsts = jnp.concatenate([dst, jnp.full((pad,), N, jnp.int32)])
    dsts = dsts.reshape(NCHUNK, CHUNK)

    al = alpha.reshape(L, 1, 1)
    be = beta.reshape(L, 1, 1)
    lns = ln_scale.reshape(L, 1, D)
    lnb = ln_bias.reshape(L, 1, D)
    bh = b_head.reshape(1, D)

    xc, y0, y1 = _enc_call(x, W_enc, Wmsg[0])
    out = None
    for i in range(L):
        a0, a1 = _sc_agg(y0, y1, srcs, dsts)
        if i < L - 1:
            xc, y0, y1 = _combine_mid_call(
                xc, a0, a1, Wself[i], Wmsg[i + 1], al[i], be[i], lns[i],
                lnb[i], residual=(i > 0))
        else:
            out = _combine_last_call(
                xc, a0, a1, Wself[i], W_head, al[i], be[i], lns[i], lnb[i],
                bh)
    return out
